# baseline (speedup 1.0000x reference)
import jax
import jax.numpy as jnp
import numpy as np
from jax import lax
from jax.experimental import pallas as pl
from jax.experimental.pallas import tpu as pltpu
from jax.experimental.pallas import tpu_sc as plsc

B, S, K, V, D = 1024, 50, 10, 1000000, 64
R = S * K
RP = 512
NCHUNK = RP // 128
NC, NS = 2, 16
NW = NC * NS
UPW = B // NW
TRASH = S
L = 16


def _body(ids_hbm, resp_hbm, tmap_hbm, table_hbm, item_out, user_out,
          idbuf, segidx, respv, cmref, tmapv, rows, segsh, segv, zbuf, ubuf,
          sem):
  sid = lax.axis_index("s")
  wid = sid * NC + lax.axis_index("c")
  base = wid * UPW

  zero16 = jnp.zeros((L,), jnp.float32)

  pltpu.sync_copy(tmap_hbm, tmapv)
  for s in range(S + 1):
    for c in range(D // L):
      zbuf[s, pl.ds(c * L, L)] = zero16
  pltpu.sync_copy(zbuf, segsh.at[sid])

  def one_user(i, carry_unused):
    b = base + i

    pltpu.sync_copy(ids_hbm.at[b], idbuf)
    pltpu.sync_copy(resp_hbm.at[b], respv)

    cps = []
    for j in range(NCHUNK):
      cps.append(pltpu.async_copy(
          table_hbm.at[idbuf.at[j]], rows.at[pl.ds(j * 128, 128)], sem))
    for cp in cps:
      cp.wait()

    carry = jnp.float32(0.0)
    for c in range(RP // L):
      rv = respv[pl.ds(c * L, L)]
      inc = plsc.cumsum(rv)
      cmref[pl.ds(c * L, L)] = carry + inc - rv
      carry = carry + jnp.sum(rv)
      tm = tmapv[pl.ds(c * L, L)]
      seg = jnp.where(rv > 0.0, tm, jnp.full((L,), TRASH, jnp.int32))
      segidx[c // 8, pl.ds((c % 8) * L, L)] = seg

    pltpu.sync_copy(rows.at[pl.ds(0, R)], item_out.at[b])

    for j in range(NCHUNK):
      pltpu.sync_copy(rows.at[pl.ds(j * 128, 128)],
                      segsh.at[sid].at[segidx.at[j]], add=True)
    pltpu.sync_copy(segsh.at[sid], segv)
    pltpu.sync_copy(zbuf, segsh.at[sid])

    acc = [zero16] * (D // L)
    for s in range(S):
      den = plsc.load_gather(cmref, [jnp.full((L,), 10 * s, jnp.int32)])
      den = jnp.maximum(den, 1.0)
      for c in range(D // L):
        acc[c] = acc[c] + segv[s, pl.ds(c * L, L)]
        ubuf[s, pl.ds(c * L, L)] = acc[c] / den

    pltpu.sync_copy(ubuf, user_out.at[b])
    return carry_unused

  lax.fori_loop(0, UPW, one_user, jnp.int32(0))


@jax.jit
def kernel(table, item_ids, responses):
  ids = item_ids.astype(jnp.int32).reshape(B, R)
  ids = jnp.pad(ids, ((0, 0), (0, RP - R))).reshape(B, NCHUNK, 128)
  respf = responses.astype(jnp.float32).reshape(B, R)
  respf = jnp.pad(respf, ((0, 0), (0, RP - R)))

  t = np.arange(RP) // K
  tmap = jnp.asarray(np.where(t < S - 1, t + 1, TRASH), dtype=jnp.int32)

  mesh = plsc.VectorSubcoreMesh(core_axis_name="c", subcore_axis_name="s")
  item_emb, user_emb = pl.kernel(
      _body,
      out_type=(
          jax.ShapeDtypeStruct((B, R, D), jnp.float32),
          jax.ShapeDtypeStruct((B, S, D), jnp.float32),
      ),
      mesh=mesh,
      compiler_params=pltpu.CompilerParams(
          needs_layout_passes=False, use_tc_tiling_on_sc=False),
      scratch_types=[
          pltpu.VMEM((NCHUNK, 128), jnp.int32),
          pltpu.VMEM((NCHUNK, 128), jnp.int32),
          pltpu.VMEM((RP,), jnp.float32),
          pltpu.VMEM((RP,), jnp.float32),
          pltpu.VMEM((RP,), jnp.int32),
          pltpu.VMEM((RP, D), jnp.float32),
          pltpu.VMEM_SHARED((NS, S + 1, D), jnp.float32),
          pltpu.VMEM((S + 1, D), jnp.float32),
          pltpu.VMEM((S + 1, D), jnp.float32),
          pltpu.VMEM((S, D), jnp.float32),
          pltpu.SemaphoreType.DMA,
      ],
  )(ids, respf, tmap, table)
  return item_emb.reshape(B, S, K, D), user_emb
